# Initial kernel scaffold; baseline (speedup 1.0000x reference)
import functools

import jax
import jax.numpy as jnp
from jax import lax
from jax.experimental import pallas as pl
from jax.experimental.pallas import tpu as pltpu
from jax.experimental.pallas import tpu_sc as plsc

N = 10000
E = 320000
DIN = 128
D = 64
NCLS = 10

NC = 2
NS = 16
NW = NC * NS
EPT = E // NW
B = 80
NBLK = EPT // B

R = 1000
NG = N // R

_mesh = plsc.VectorSubcoreMesh(
    core_axis_name="c", subcore_axis_name="s", num_cores=NC, num_subcores=NS)


def _deg_body(dst3_hbm, z1_hbm, out_hbm, dst_idx, ones_v, deg_sh, sem):
    c = lax.axis_index("c")
    s = lax.axis_index("s")
    chunk = c * NS + s
    pltpu.sync_copy(dst3_hbm.at[chunk], dst_idx)
    for k in range(B // 16):
        ones_v[pl.ds(k * 16, 16)] = jnp.ones((16,), jnp.float32)

    @pl.when(s == 0)
    def _():
        pltpu.sync_copy(z1_hbm, deg_sh)

    plsc.subcore_barrier()

    def blk(j, carry):
        pltpu.sync_copy(ones_v, deg_sh.at[dst_idx.at[j]], add=True)
        return carry

    lax.fori_loop(0, NBLK, blk, 0)
    plsc.subcore_barrier()

    @pl.when(s < 10)
    def _():
        pltpu.sync_copy(deg_sh.at[pl.ds(s * 1000, 1000)],
                        out_hbm.at[c, pl.ds(s * 1000, 1000)])


def _sc_deg(dst3, z1):
    return pl.kernel(
        _deg_body,
        out_type=jax.ShapeDtypeStruct((NC, N), jnp.float32),
        mesh=_mesh,
        scratch_types=[
            pltpu.VMEM((NBLK, B), jnp.int32),
            pltpu.VMEM((B,), jnp.float32),
            pltpu.VMEM_SHARED((N,), jnp.float32),
            pltpu.SemaphoreType.DMA,
        ],
    )(dst3, z1)


def _gcn_body(g_hbm, src3_hbm, dst3_hbm, z64_hbm, out_hbm,
              src_idx, dst_idx, rows, acc_sh, gsem):
    c = lax.axis_index("c")
    s = lax.axis_index("s")
    chunk = c * NS + s
    pltpu.sync_copy(src3_hbm.at[chunk], src_idx)
    pltpu.sync_copy(dst3_hbm.at[chunk], dst_idx)

    @pl.when(s == 0)
    def _():
        pltpu.sync_copy(z64_hbm, acc_sh)

    plsc.subcore_barrier()

    def blk(j, carry):
        pltpu.async_copy(g_hbm.at[src_idx.at[j]], rows, gsem).wait()
        pltpu.sync_copy(rows, acc_sh.at[dst_idx.at[j]], add=True)
        return carry

    lax.fori_loop(0, NBLK, blk, 0)
    plsc.subcore_barrier()
    rows_per = N // NS
    pltpu.sync_copy(acc_sh.at[pl.ds(s * rows_per, rows_per)],
                    out_hbm.at[c, pl.ds(s * rows_per, rows_per)])


def _sc_gcn(g, src3, dst3, z64):
    return pl.kernel(
        _gcn_body,
        out_type=jax.ShapeDtypeStruct((NC, N, D), jnp.float32),
        mesh=_mesh,
        scratch_types=[
            pltpu.VMEM((NBLK, B), jnp.int32),
            pltpu.VMEM((NBLK, B), jnp.int32),
            pltpu.VMEM((B, D), jnp.float32),
            pltpu.VMEM_SHARED((N, D), jnp.float32),
            pltpu.SemaphoreType.DMA,
        ],
    )(g, src3, dst3, z64)


def _gat_body(hh_hbm, p_hbm, q_hbm, pmax_hbm, src3_hbm, dst3_hbm,
              z64_hbm, z1_hbm, acc_out, s_out,
              src_idx, dst_idx, rows, w_buf, p_v, q_v, pm_v, acc_sh, s_sh,
              gsem):
    c = lax.axis_index("c")
    s = lax.axis_index("s")
    chunk = c * NS + s
    pltpu.sync_copy(src3_hbm.at[chunk], src_idx)
    pltpu.sync_copy(dst3_hbm.at[chunk], dst_idx)
    pltpu.sync_copy(p_hbm, p_v)
    pltpu.sync_copy(q_hbm, q_v)
    pltpu.sync_copy(pmax_hbm, pm_v)

    @pl.when(s == 0)
    def _():
        pltpu.sync_copy(z64_hbm, acc_sh)
        pltpu.sync_copy(z1_hbm, s_sh)

    plsc.subcore_barrier()
    pmv = pm_v[...]

    def blk(j, carry):
        pltpu.async_copy(hh_hbm.at[src_idx.at[j]], rows, gsem).wait()
        for k in range(B // 16):
            si = src_idx[j, pl.ds(k * 16, 16)]
            di = dst_idx[j, pl.ds(k * 16, 16)]
            pv = plsc.load_gather(p_v, [si])
            qv = plsc.load_gather(q_v, [di])
            z = pv + qv
            e = jnp.maximum(z, 0.2 * z)
            zu = pmv + qv
            u = jnp.maximum(zu, 0.2 * zu)
            w_buf[pl.ds(k * 16, 16)] = jnp.exp(e - u)

        def wb(i, carry2):
            bw = plsc.load_gather(w_buf, [jnp.full((16,), i, jnp.int32)])
            for t in range(D // 16):
                rows[i, pl.ds(t * 16, 16)] = rows[i, pl.ds(t * 16, 16)] * bw
            return carry2

        lax.fori_loop(0, B, wb, 0)
        pltpu.sync_copy(rows, acc_sh.at[dst_idx.at[j]], add=True)
        pltpu.sync_copy(w_buf, s_sh.at[dst_idx.at[j]], add=True)
        return carry

    lax.fori_loop(0, NBLK, blk, 0)
    plsc.subcore_barrier()
    rows_per = N // NS
    pltpu.sync_copy(acc_sh.at[pl.ds(s * rows_per, rows_per)],
                    acc_out.at[c, pl.ds(s * rows_per, rows_per)])

    @pl.when(s < 10)
    def _():
        pltpu.sync_copy(s_sh.at[pl.ds(s * 1000, 1000)],
                        s_out.at[c, pl.ds(s * 1000, 1000)])


def _sc_gat(hh, p, q, pmax, src3, dst3, z64, z1):
    return pl.kernel(
        _gat_body,
        out_type=(jax.ShapeDtypeStruct((NC, N, D), jnp.float32),
                  jax.ShapeDtypeStruct((NC, N), jnp.float32)),
        mesh=_mesh,
        scratch_types=[
            pltpu.VMEM((NBLK, B), jnp.int32),
            pltpu.VMEM((NBLK, B), jnp.int32),
            pltpu.VMEM((B, D), jnp.float32),
            pltpu.VMEM((B,), jnp.float32),
            pltpu.VMEM((N,), jnp.float32),
            pltpu.VMEM((N,), jnp.float32),
            pltpu.VMEM((16,), jnp.float32),
            pltpu.VMEM_SHARED((N, D), jnp.float32),
            pltpu.VMEM_SHARED((N,), jnp.float32),
            pltpu.SemaphoreType.DMA,
        ],
    )(hh, p, q, pmax, src3, dst3, z64, z1)


def _tc1_body(x_ref, w1_ref, da_ref, db_ref, g1_ref, dinv_ref):
    deg = da_ref[...] + db_ref[...] + 1.0
    dv = lax.rsqrt(deg)
    g1_ref[...] = jnp.dot(x_ref[...], w1_ref[...],
                          preferred_element_type=jnp.float32) * dv
    dinv_ref[...] = dv


def _tc1(x, W1, dA, dB):
    return pl.pallas_call(
        _tc1_body,
        grid=(NG,),
        in_specs=[
            pl.BlockSpec((R, DIN), lambda i: (i, 0)),
            pl.BlockSpec((DIN, D), lambda i: (0, 0)),
            pl.BlockSpec((R, 1), lambda i: (i, 0)),
            pl.BlockSpec((R, 1), lambda i: (i, 0)),
        ],
        out_specs=[
            pl.BlockSpec((R, D), lambda i: (i, 0)),
            pl.BlockSpec((R, 1), lambda i: (i, 0)),
        ],
        out_shape=[
            jax.ShapeDtypeStruct((N, D), jnp.float32),
            jax.ShapeDtypeStruct((N, 1), jnp.float32),
        ],
    )(x, W1, dA, dB)


def _tc2_body(accp_ref, g1_ref, dinv_ref, b1_ref, w2_ref, as_ref, ad_ref,
              hh_ref, p_ref, q_ref, pmax_ref):
    i = pl.program_id(0)
    acc = accp_ref[0] + accp_ref[1] + g1_ref[...]
    h1 = jnp.maximum(dinv_ref[...] * acc + b1_ref[...], 0.0)
    hh = jnp.dot(h1, w2_ref[...], preferred_element_type=jnp.float32)
    hh_ref[...] = hh
    p = jnp.dot(hh, as_ref[...], preferred_element_type=jnp.float32)
    q = jnp.dot(hh, ad_ref[...], preferred_element_type=jnp.float32)
    p_ref[...] = p
    q_ref[...] = q
    pb = jnp.max(p)

    @pl.when(i == 0)
    def _():
        pmax_ref[0, 0] = pb

    @pl.when(i > 0)
    def _():
        pmax_ref[0, 0] = jnp.maximum(pmax_ref[0, 0], pb)


def _tc2(accP, g1, dinv, b1, W2, aS, aD):
    return pl.pallas_call(
        _tc2_body,
        grid=(NG,),
        in_specs=[
            pl.BlockSpec((NC, R, D), lambda i: (0, i, 0)),
            pl.BlockSpec((R, D), lambda i: (i, 0)),
            pl.BlockSpec((R, 1), lambda i: (i, 0)),
            pl.BlockSpec((1, D), lambda i: (0, 0)),
            pl.BlockSpec((D, D), lambda i: (0, 0)),
            pl.BlockSpec((D, 1), lambda i: (0, 0)),
            pl.BlockSpec((D, 1), lambda i: (0, 0)),
        ],
        out_specs=[
            pl.BlockSpec((R, D), lambda i: (i, 0)),
            pl.BlockSpec((R, 1), lambda i: (i, 0)),
            pl.BlockSpec((R, 1), lambda i: (i, 0)),
            pl.BlockSpec((1, 1), lambda i: (0, 0)),
        ],
        out_shape=[
            jax.ShapeDtypeStruct((N, D), jnp.float32),
            jax.ShapeDtypeStruct((N, 1), jnp.float32),
            jax.ShapeDtypeStruct((N, 1), jnp.float32),
            jax.ShapeDtypeStruct((1, 1), jnp.float32),
        ],
    )(accP, g1, dinv, b1, W2, aS, aD)


def _tc3_body(accp_ref, sp_ref, hh_ref, p_ref, q_ref, pmax_ref, dinv_ref,
              b2_ref, w3_ref, g3_ref):
    p = p_ref[...]
    q = q_ref[...]
    z = p + q
    e_self = jnp.maximum(z, 0.2 * z)
    zu = pmax_ref[0, 0] + q
    u = jnp.maximum(zu, 0.2 * zu)
    w_self = jnp.exp(e_self - u)
    den = sp_ref[0] + sp_ref[1] + w_self + 1e-16
    num = accp_ref[0] + accp_ref[1] + w_self * hh_ref[...]
    h2 = jnp.maximum(num / den + b2_ref[...], 0.0)
    g3_ref[...] = jnp.dot(h2, w3_ref[...],
                          preferred_element_type=jnp.float32) * dinv_ref[...]


def _tc3(accP, sP, hh, p, q, pmax, dinv, b2, W3):
    return pl.pallas_call(
        _tc3_body,
        grid=(NG,),
        in_specs=[
            pl.BlockSpec((NC, R, D), lambda i: (0, i, 0)),
            pl.BlockSpec((NC, R, 1), lambda i: (0, i, 0)),
            pl.BlockSpec((R, D), lambda i: (i, 0)),
            pl.BlockSpec((R, 1), lambda i: (i, 0)),
            pl.BlockSpec((R, 1), lambda i: (i, 0)),
            pl.BlockSpec((1, 1), lambda i: (0, 0)),
            pl.BlockSpec((R, 1), lambda i: (i, 0)),
            pl.BlockSpec((1, D), lambda i: (0, 0)),
            pl.BlockSpec((D, D), lambda i: (0, 0)),
        ],
        out_specs=[pl.BlockSpec((R, D), lambda i: (i, 0))],
        out_shape=[jax.ShapeDtypeStruct((N, D), jnp.float32)],
    )(accP, sP, hh, p, q, pmax, dinv, b2, W3)


def _tc4_body(accp_ref, g3_ref, dinv_ref, b3_ref, wo_ref, bo_ref,
              wb1_ref, bb1_ref, wb2_ref, bb2_ref,
              opt_ref, bt_ref, ge_ref):
    i = pl.program_id(0)
    acc = accp_ref[0] + accp_ref[1] + g3_ref[...]
    h3 = jnp.maximum(dinv_ref[...] * acc + b3_ref[...], 0.0)
    opt_ref[...] = jnp.dot(h3, wo_ref[...],
                           preferred_element_type=jnp.float32) + bo_ref[...]
    t = jnp.maximum(jnp.dot(h3, wb1_ref[...],
                            preferred_element_type=jnp.float32) + bb1_ref[...],
                    0.0)
    bt_ref[...] = jax.nn.sigmoid(
        jnp.dot(t, wb2_ref[...], preferred_element_type=jnp.float32)
        + bb2_ref[...])
    tot = jnp.sum(h3, axis=0, keepdims=True)

    @pl.when(i == 0)
    def _():
        ge_ref[...] = tot

    @pl.when(i > 0)
    def _():
        ge_ref[...] = ge_ref[...] + tot

    @pl.when(i == NG - 1)
    def _():
        ge_ref[...] = ge_ref[...] * (1.0 / N)


def _tc4(accP, g3, dinv, b3, Wo, bo, Wb1, bb1, Wb2, bb2):
    return pl.pallas_call(
        _tc4_body,
        grid=(NG,),
        in_specs=[
            pl.BlockSpec((NC, R, D), lambda i: (0, i, 0)),
            pl.BlockSpec((R, D), lambda i: (i, 0)),
            pl.BlockSpec((R, 1), lambda i: (i, 0)),
            pl.BlockSpec((1, D), lambda i: (0, 0)),
            pl.BlockSpec((D, NCLS), lambda i: (0, 0)),
            pl.BlockSpec((1, NCLS), lambda i: (0, 0)),
            pl.BlockSpec((D, 32), lambda i: (0, 0)),
            pl.BlockSpec((1, 32), lambda i: (0, 0)),
            pl.BlockSpec((32, 1), lambda i: (0, 0)),
            pl.BlockSpec((1, 1), lambda i: (0, 0)),
        ],
        out_specs=[
            pl.BlockSpec((R, NCLS), lambda i: (i, 0)),
            pl.BlockSpec((R, 1), lambda i: (i, 0)),
            pl.BlockSpec((1, D), lambda i: (0, 0)),
        ],
        out_shape=[
            jax.ShapeDtypeStruct((N, NCLS), jnp.float32),
            jax.ShapeDtypeStruct((N, 1), jnp.float32),
            jax.ShapeDtypeStruct((1, D), jnp.float32),
        ],
    )(accP, g3, dinv, b3, Wo, bo, Wb1, bb1, Wb2, bb2)


def kernel(x, edge_index, W1, b1, W2, a_src, a_dst, b2, W3, b3, Wo, bo,
           Wb1, bb1, Wb2, bb2):
    src3 = edge_index[0].reshape(NW, NBLK, B)
    dst3 = edge_index[1].reshape(NW, NBLK, B)
    z64 = jnp.zeros((N, D), jnp.float32)
    z1 = jnp.zeros((N,), jnp.float32)

    degP = _sc_deg(dst3, z1)
    dA = degP[0].reshape(N, 1)
    dB = degP[1].reshape(N, 1)

    g1, dinv = _tc1(x, W1, dA, dB)
    acc1 = _sc_gcn(g1, src3, dst3, z64)
    hh, p, q, pmax = _tc2(acc1, g1, dinv, b1.reshape(1, D), W2,
                          a_src.reshape(D, 1), a_dst.reshape(D, 1))

    pmax16 = jnp.broadcast_to(pmax.reshape(1), (16,))
    acc2, s2 = _sc_gat(hh, p.reshape(N), q.reshape(N), pmax16,
                       src3, dst3, z64, z1)
    g3 = _tc3(acc2, s2.reshape(NC, N, 1), hh, p, q, pmax, dinv,
              b2.reshape(1, D), W3)[0]

    acc3 = _sc_gcn(g3, src3, dst3, z64)
    opt, bt, ge = _tc4(acc3, g3, dinv, b3.reshape(1, D), Wo,
                       bo.reshape(1, NCLS), Wb1, bb1.reshape(1, 32),
                       Wb2, bb2.reshape(1, 1))
    return opt, bt, ge.reshape(D)


# speedup vs baseline: 23.6490x; 23.6490x over previous
import functools

import jax
import jax.numpy as jnp
from jax import lax
from jax.experimental import pallas as pl
from jax.experimental.pallas import tpu as pltpu
from jax.experimental.pallas import tpu_sc as plsc

N = 10000
E = 320000
DIN = 128
D = 64
NCLS = 10

NC = 2
NS = 16
NW = NC * NS
EPT = E // NW
B = 80
NBLK = EPT // B

R = 1000
NG = N // R

_mesh = plsc.VectorSubcoreMesh(
    core_axis_name="c", subcore_axis_name="s", num_cores=NC, num_subcores=NS)
_sc_params = pltpu.CompilerParams(use_tc_tiling_on_sc=False,
                                  needs_layout_passes=False)


def _zero_vec(buf, nv):
    def zb(t, carry):
        buf[pl.ds(t * 16, 16)] = jnp.zeros((16,), jnp.float32)
        return carry
    lax.fori_loop(0, nv, zb, 0)


def _deg_body(dst3_hbm, out_hbm, dst_idx, ones_v, buf1, deg_sh, sem):
    c = lax.axis_index("c")
    s = lax.axis_index("s")
    chunk = c * NS + s
    pltpu.sync_copy(dst3_hbm.at[chunk], dst_idx)
    for k in range(B // 16):
        ones_v[pl.ds(k * 16, 16)] = jnp.ones((16,), jnp.float32)
    _zero_vec(buf1, 63)

    @pl.when(s < 10)
    def _():
        pltpu.sync_copy(buf1.at[pl.ds(0, 1000)],
                        deg_sh.at[pl.ds(s * 1000, 1000)])

    plsc.subcore_barrier()

    def blk(j, carry):
        pltpu.sync_copy(ones_v, deg_sh.at[dst_idx.at[j]], add=True)
        return carry

    lax.fori_loop(0, NBLK, blk, 0)
    plsc.subcore_barrier()

    @pl.when(s < 10)
    def _():
        pltpu.sync_copy(deg_sh.at[pl.ds(s * 1000, 1000)],
                        buf1.at[pl.ds(0, 1000)])
        pltpu.sync_copy(buf1.at[pl.ds(0, 1000)],
                        out_hbm.at[pl.ds(c * N + s * 1000, 1000)])


def _sc_deg(dst3):
    return pl.kernel(
        _deg_body,
        out_type=jax.ShapeDtypeStruct((NC * N,), jnp.float32),
        mesh=_mesh,
        compiler_params=_sc_params,
        scratch_types=[
            pltpu.VMEM((NBLK, B), jnp.int32),
            pltpu.VMEM((B,), jnp.float32),
            pltpu.VMEM((1008,), jnp.float32),
            pltpu.VMEM_SHARED((N,), jnp.float32),
            pltpu.SemaphoreType.DMA,
        ],
    )(dst3)


def _zero_rows(buf, nrows):
    def zb(r, carry):
        for t in range(D // 16):
            buf[r, pl.ds(t * 16, 16)] = jnp.zeros((16,), jnp.float32)
        return carry
    lax.fori_loop(0, nrows, zb, 0)


def _acc_init(acc_sh, bounce, s):
    _zero_rows(bounce, 125)

    def zc(ch, carry):
        pltpu.sync_copy(bounce, acc_sh.at[pl.ds(s * 625 + ch * 125, 125)])
        return carry

    lax.fori_loop(0, 5, zc, 0)


def _acc_out(acc_sh, bounce, out_hbm, c, s):
    def oc(ch, carry):
        r0 = s * 625 + ch * 125
        pltpu.sync_copy(acc_sh.at[pl.ds(r0, 125)], bounce)
        pltpu.sync_copy(bounce, out_hbm.at[c, pl.ds(r0, 125)])
        return carry

    lax.fori_loop(0, 5, oc, 0)


def _gcn_body(g_hbm, src3_hbm, dst3_hbm, out_hbm,
              src_idx, dst_idx, rows, big_buf, acc_sh, gsem):
    c = lax.axis_index("c")
    s = lax.axis_index("s")
    chunk = c * NS + s
    pltpu.sync_copy(src3_hbm.at[chunk], src_idx)
    pltpu.sync_copy(dst3_hbm.at[chunk], dst_idx)
    _acc_init(acc_sh, big_buf, s)
    plsc.subcore_barrier()

    def blk(j, carry):
        pltpu.async_copy(g_hbm.at[src_idx.at[j]], rows, gsem).wait()
        pltpu.sync_copy(rows, acc_sh.at[dst_idx.at[j]], add=True)
        return carry

    lax.fori_loop(0, NBLK, blk, 0)
    plsc.subcore_barrier()
    _acc_out(acc_sh, big_buf, out_hbm, c, s)


def _sc_gcn(g, src3, dst3):
    return pl.kernel(
        _gcn_body,
        out_type=jax.ShapeDtypeStruct((NC, N, D), jnp.float32),
        mesh=_mesh,
        compiler_params=_sc_params,
        scratch_types=[
            pltpu.VMEM((NBLK, B), jnp.int32),
            pltpu.VMEM((NBLK, B), jnp.int32),
            pltpu.VMEM((B, D), jnp.float32),
            pltpu.VMEM((125, D), jnp.float32),
            pltpu.VMEM_SHARED((N, D), jnp.float32),
            pltpu.SemaphoreType.DMA,
        ],
    )(g, src3, dst3)


def _gat_body(hh_hbm, p_hbm, q_hbm, pmax_hbm, src3_hbm, dst3_hbm,
              acc_out, s_out,
              src_idx, dst_idx, rows, w_buf, p_v, q_v, pm_v, buf1,
              big_buf, acc_sh, s_sh, gsem):
    c = lax.axis_index("c")
    s = lax.axis_index("s")
    chunk = c * NS + s
    pltpu.sync_copy(src3_hbm.at[chunk], src_idx)
    pltpu.sync_copy(dst3_hbm.at[chunk], dst_idx)
    pltpu.sync_copy(p_hbm, p_v)
    pltpu.sync_copy(q_hbm, q_v)
    pltpu.sync_copy(pmax_hbm, pm_v)
    _acc_init(acc_sh, big_buf, s)
    _zero_vec(buf1, 63)

    @pl.when(s < 10)
    def _():
        pltpu.sync_copy(buf1.at[pl.ds(0, 1000)],
                        s_sh.at[pl.ds(s * 1000, 1000)])

    plsc.subcore_barrier()
    pmv = pm_v[...]

    def blk(j, carry):
        pltpu.async_copy(hh_hbm.at[src_idx.at[j]], rows, gsem).wait()
        for k in range(B // 16):
            si = src_idx[j, pl.ds(k * 16, 16)]
            di = dst_idx[j, pl.ds(k * 16, 16)]
            pv = plsc.load_gather(p_v, [si])
            qv = plsc.load_gather(q_v, [di])
            z = pv + qv
            e = jnp.maximum(z, 0.2 * z)
            zu = pmv + qv
            u = jnp.maximum(zu, 0.2 * zu)
            w_buf[pl.ds(k * 16, 16)] = jnp.exp(e - u)

        def wb(i, carry2):
            bw = plsc.load_gather(w_buf, [jnp.full((16,), i, jnp.int32)])
            for t in range(D // 16):
                rows[i, pl.ds(t * 16, 16)] = rows[i, pl.ds(t * 16, 16)] * bw
            return carry2

        lax.fori_loop(0, B, wb, 0)
        pltpu.sync_copy(rows, acc_sh.at[dst_idx.at[j]], add=True)
        pltpu.sync_copy(w_buf, s_sh.at[dst_idx.at[j]], add=True)
        return carry

    lax.fori_loop(0, NBLK, blk, 0)
    plsc.subcore_barrier()
    _acc_out(acc_sh, big_buf, acc_out, c, s)

    @pl.when(s < 10)
    def _():
        pltpu.sync_copy(s_sh.at[pl.ds(s * 1000, 1000)],
                        buf1.at[pl.ds(0, 1000)])
        pltpu.sync_copy(buf1.at[pl.ds(0, 1000)],
                        s_out.at[pl.ds(c * N + s * 1000, 1000)])


def _sc_gat(hh, p, q, pmax, src3, dst3):
    return pl.kernel(
        _gat_body,
        out_type=(jax.ShapeDtypeStruct((NC, N, D), jnp.float32),
                  jax.ShapeDtypeStruct((NC * N,), jnp.float32)),
        mesh=_mesh,
        compiler_params=_sc_params,
        scratch_types=[
            pltpu.VMEM((NBLK, B), jnp.int32),
            pltpu.VMEM((NBLK, B), jnp.int32),
            pltpu.VMEM((B, D), jnp.float32),
            pltpu.VMEM((B,), jnp.float32),
            pltpu.VMEM((N,), jnp.float32),
            pltpu.VMEM((N,), jnp.float32),
            pltpu.VMEM((16,), jnp.float32),
            pltpu.VMEM((1008,), jnp.float32),
            pltpu.VMEM((125, D), jnp.float32),
            pltpu.VMEM_SHARED((N, D), jnp.float32),
            pltpu.VMEM_SHARED((N,), jnp.float32),
            pltpu.SemaphoreType.DMA,
        ],
    )(hh, p, q, pmax, src3, dst3)


def _tc1_body(x_ref, w1_ref, da_ref, db_ref, g1_ref, dinv_ref):
    deg = da_ref[...] + db_ref[...] + 1.0
    dv = lax.rsqrt(deg)
    g1_ref[...] = jnp.dot(x_ref[...], w1_ref[...],
                          preferred_element_type=jnp.float32) * dv
    dinv_ref[...] = dv


def _tc1(x, W1, dA, dB):
    return pl.pallas_call(
        _tc1_body,
        grid=(NG,),
        in_specs=[
            pl.BlockSpec((R, DIN), lambda i: (i, 0)),
            pl.BlockSpec((DIN, D), lambda i: (0, 0)),
            pl.BlockSpec((R, 1), lambda i: (i, 0)),
            pl.BlockSpec((R, 1), lambda i: (i, 0)),
        ],
        out_specs=[
            pl.BlockSpec((R, D), lambda i: (i, 0)),
            pl.BlockSpec((R, 1), lambda i: (i, 0)),
        ],
        out_shape=[
            jax.ShapeDtypeStruct((N, D), jnp.float32),
            jax.ShapeDtypeStruct((N, 1), jnp.float32),
        ],
    )(x, W1, dA, dB)


def _tc2_body(accp_ref, g1_ref, dinv_ref, b1_ref, w2_ref, as_ref, ad_ref,
              hh_ref, p_ref, q_ref, pmax_ref):
    i = pl.program_id(0)
    acc = accp_ref[0] + accp_ref[1] + g1_ref[...]
    h1 = jnp.maximum(dinv_ref[...] * acc + b1_ref[...], 0.0)
    hh = jnp.dot(h1, w2_ref[...], preferred_element_type=jnp.float32)
    hh_ref[...] = hh
    p = jnp.dot(hh, as_ref[...], preferred_element_type=jnp.float32)
    q = jnp.dot(hh, ad_ref[...], preferred_element_type=jnp.float32)
    p_ref[...] = p
    q_ref[...] = q
    pb = jnp.max(p, axis=(0, 1), keepdims=True)

    @pl.when(i == 0)
    def _():
        pmax_ref[...] = pb

    @pl.when(i > 0)
    def _():
        pmax_ref[...] = jnp.maximum(pmax_ref[...], pb)


def _tc2(accP, g1, dinv, b1, W2, aS, aD):
    return pl.pallas_call(
        _tc2_body,
        grid=(NG,),
        in_specs=[
            pl.BlockSpec((NC, R, D), lambda i: (0, i, 0)),
            pl.BlockSpec((R, D), lambda i: (i, 0)),
            pl.BlockSpec((R, 1), lambda i: (i, 0)),
            pl.BlockSpec((1, D), lambda i: (0, 0)),
            pl.BlockSpec((D, D), lambda i: (0, 0)),
            pl.BlockSpec((D, 1), lambda i: (0, 0)),
            pl.BlockSpec((D, 1), lambda i: (0, 0)),
        ],
        out_specs=[
            pl.BlockSpec((R, D), lambda i: (i, 0)),
            pl.BlockSpec((R, 1), lambda i: (i, 0)),
            pl.BlockSpec((R, 1), lambda i: (i, 0)),
            pl.BlockSpec((1, 1), lambda i: (0, 0)),
        ],
        out_shape=[
            jax.ShapeDtypeStruct((N, D), jnp.float32),
            jax.ShapeDtypeStruct((N, 1), jnp.float32),
            jax.ShapeDtypeStruct((N, 1), jnp.float32),
            jax.ShapeDtypeStruct((1, 1), jnp.float32),
        ],
    )(accP, g1, dinv, b1, W2, aS, aD)


def _tc3_body(accp_ref, sp_ref, hh_ref, p_ref, q_ref, pmax_ref, dinv_ref,
              b2_ref, w3_ref, g3_ref):
    p = p_ref[...]
    q = q_ref[...]
    z = p + q
    e_self = jnp.maximum(z, 0.2 * z)
    zu = pmax_ref[0, 0] + q
    u = jnp.maximum(zu, 0.2 * zu)
    w_self = jnp.exp(e_self - u)
    den = sp_ref[0] + sp_ref[1] + w_self + 1e-16
    num = accp_ref[0] + accp_ref[1] + w_self * hh_ref[...]
    h2 = jnp.maximum(num / den + b2_ref[...], 0.0)
    g3_ref[...] = jnp.dot(h2, w3_ref[...],
                          preferred_element_type=jnp.float32) * dinv_ref[...]


def _tc3(accP, sP, hh, p, q, pmax, dinv, b2, W3):
    return pl.pallas_call(
        _tc3_body,
        grid=(NG,),
        in_specs=[
            pl.BlockSpec((NC, R, D), lambda i: (0, i, 0)),
            pl.BlockSpec((NC, R, 1), lambda i: (0, i, 0)),
            pl.BlockSpec((R, D), lambda i: (i, 0)),
            pl.BlockSpec((R, 1), lambda i: (i, 0)),
            pl.BlockSpec((R, 1), lambda i: (i, 0)),
            pl.BlockSpec((1, 1), lambda i: (0, 0)),
            pl.BlockSpec((R, 1), lambda i: (i, 0)),
            pl.BlockSpec((1, D), lambda i: (0, 0)),
            pl.BlockSpec((D, D), lambda i: (0, 0)),
        ],
        out_specs=[pl.BlockSpec((R, D), lambda i: (i, 0))],
        out_shape=[jax.ShapeDtypeStruct((N, D), jnp.float32)],
    )(accP, sP, hh, p, q, pmax, dinv, b2, W3)


def _tc4_body(accp_ref, g3_ref, dinv_ref, b3_ref, wo_ref, bo_ref,
              wb1_ref, bb1_ref, wb2_ref, bb2_ref,
              opt_ref, bt_ref, ge_ref):
    i = pl.program_id(0)
    acc = accp_ref[0] + accp_ref[1] + g3_ref[...]
    h3 = jnp.maximum(dinv_ref[...] * acc + b3_ref[...], 0.0)
    opt_ref[...] = jnp.dot(h3, wo_ref[...],
                           preferred_element_type=jnp.float32) + bo_ref[...]
    t = jnp.maximum(jnp.dot(h3, wb1_ref[...],
                            preferred_element_type=jnp.float32) + bb1_ref[...],
                    0.0)
    bt_ref[...] = jax.nn.sigmoid(
        jnp.dot(t, wb2_ref[...], preferred_element_type=jnp.float32)
        + bb2_ref[...])
    tot = jnp.sum(h3, axis=0, keepdims=True)

    @pl.when(i == 0)
    def _():
        ge_ref[...] = tot

    @pl.when(i > 0)
    def _():
        ge_ref[...] = ge_ref[...] + tot

    @pl.when(i == NG - 1)
    def _():
        ge_ref[...] = ge_ref[...] * (1.0 / N)


def _tc4(accP, g3, dinv, b3, Wo, bo, Wb1, bb1, Wb2, bb2):
    return pl.pallas_call(
        _tc4_body,
        grid=(NG,),
        in_specs=[
            pl.BlockSpec((NC, R, D), lambda i: (0, i, 0)),
            pl.BlockSpec((R, D), lambda i: (i, 0)),
            pl.BlockSpec((R, 1), lambda i: (i, 0)),
            pl.BlockSpec((1, D), lambda i: (0, 0)),
            pl.BlockSpec((D, NCLS), lambda i: (0, 0)),
            pl.BlockSpec((1, NCLS), lambda i: (0, 0)),
            pl.BlockSpec((D, 32), lambda i: (0, 0)),
            pl.BlockSpec((1, 32), lambda i: (0, 0)),
            pl.BlockSpec((32, 1), lambda i: (0, 0)),
            pl.BlockSpec((1, 1), lambda i: (0, 0)),
        ],
        out_specs=[
            pl.BlockSpec((R, NCLS), lambda i: (i, 0)),
            pl.BlockSpec((R, 1), lambda i: (i, 0)),
            pl.BlockSpec((1, D), lambda i: (0, 0)),
        ],
        out_shape=[
            jax.ShapeDtypeStruct((N, NCLS), jnp.float32),
            jax.ShapeDtypeStruct((N, 1), jnp.float32),
            jax.ShapeDtypeStruct((1, D), jnp.float32),
        ],
    )(accP, g3, dinv, b3, Wo, bo, Wb1, bb1, Wb2, bb2)


def kernel(x, edge_index, W1, b1, W2, a_src, a_dst, b2, W3, b3, Wo, bo,
           Wb1, bb1, Wb2, bb2):
    src3 = edge_index[0].reshape(NW, NBLK, B)
    dst3 = edge_index[1].reshape(NW, NBLK, B)

    degP = _sc_deg(dst3).reshape(NC, N)
    dA = degP[0].reshape(N, 1)
    dB = degP[1].reshape(N, 1)

    g1, dinv = _tc1(x, W1, dA, dB)
    acc1 = _sc_gcn(g1, src3, dst3)
    hh, p, q, pmax = _tc2(acc1, g1, dinv, b1.reshape(1, D), W2,
                          a_src.reshape(D, 1), a_dst.reshape(D, 1))

    pmax16 = jnp.broadcast_to(pmax.reshape(1), (16,))
    acc2, s2 = _sc_gat(hh, p.reshape(N), q.reshape(N), pmax16,
                       src3, dst3)
    g3 = _tc3(acc2, s2.reshape(NC, N, 1), hh, p, q, pmax, dinv,
              b2.reshape(1, D), W3)[0]

    acc3 = _sc_gcn(g3, src3, dst3)
    opt, bt, ge = _tc4(acc3, g3, dinv, b3.reshape(1, D), Wo,
                       bo.reshape(1, NCLS), Wb1, bb1.reshape(1, 32),
                       Wb2, bb2.reshape(1, 1))
    return opt, bt, ge.reshape(D)


# speedup vs baseline: 37.8655x; 1.6011x over previous
import functools

import jax
import jax.numpy as jnp
from jax import lax
from jax.experimental import pallas as pl
from jax.experimental.pallas import tpu as pltpu
from jax.experimental.pallas import tpu_sc as plsc

N = 10000
E = 320000
DIN = 128
D = 64
NCLS = 10

NC = 2
NS = 16
NW = NC * NS
EPT = E // NW
B = 80
NBLK = EPT // B

R = 1000
NG = N // R

_mesh = plsc.VectorSubcoreMesh(
    core_axis_name="c", subcore_axis_name="s", num_cores=NC, num_subcores=NS)
_sc_params = pltpu.CompilerParams(use_tc_tiling_on_sc=False,
                                  needs_layout_passes=False)


def _zero_vec(buf, nv):
    def zb(t, carry):
        buf[pl.ds(t * 16, 16)] = jnp.zeros((16,), jnp.float32)
        return carry
    lax.fori_loop(0, nv, zb, 0)


def _deg_body(dst3_hbm, out_hbm, dst_idx, ones_v, buf1, deg_sh, sem):
    c = lax.axis_index("c")
    s = lax.axis_index("s")
    chunk = c * NS + s
    pltpu.sync_copy(dst3_hbm.at[chunk], dst_idx)
    for k in range(B // 16):
        ones_v[pl.ds(k * 16, 16)] = jnp.ones((16,), jnp.float32)
    _zero_vec(buf1, 63)

    @pl.when(s < 10)
    def _():
        pltpu.sync_copy(buf1.at[pl.ds(0, 1000)],
                        deg_sh.at[pl.ds(s * 1000, 1000)])

    plsc.subcore_barrier()

    def blk(j, carry):
        pltpu.sync_copy(ones_v, deg_sh.at[dst_idx.at[j]], add=True)
        return carry

    lax.fori_loop(0, NBLK, blk, 0)
    plsc.subcore_barrier()

    @pl.when(s < 10)
    def _():
        pltpu.sync_copy(deg_sh.at[pl.ds(s * 1000, 1000)],
                        buf1.at[pl.ds(0, 1000)])
        pltpu.sync_copy(buf1.at[pl.ds(0, 1000)],
                        out_hbm.at[pl.ds(c * N + s * 1000, 1000)])


def _sc_deg(dst3):
    return pl.kernel(
        _deg_body,
        out_type=jax.ShapeDtypeStruct((NC * N,), jnp.float32),
        mesh=_mesh,
        compiler_params=_sc_params,
        scratch_types=[
            pltpu.VMEM((NBLK, B), jnp.int32),
            pltpu.VMEM((B,), jnp.float32),
            pltpu.VMEM((1008,), jnp.float32),
            pltpu.VMEM_SHARED((N,), jnp.float32),
            pltpu.SemaphoreType.DMA,
        ],
    )(dst3)


def _zero_rows(buf, nrows):
    def zb(r, carry):
        for t in range(D // 16):
            buf[r, pl.ds(t * 16, 16)] = jnp.zeros((16,), jnp.float32)
        return carry
    lax.fori_loop(0, nrows, zb, 0)


def _acc_init(acc_sh, bounce, s):
    _zero_rows(bounce, 125)

    def zc(ch, carry):
        pltpu.sync_copy(bounce, acc_sh.at[pl.ds(s * 625 + ch * 125, 125)])
        return carry

    lax.fori_loop(0, 5, zc, 0)


def _acc_out(acc_sh, bounce, out_hbm, c, s):
    def oc(ch, carry):
        r0 = s * 625 + ch * 125
        pltpu.sync_copy(acc_sh.at[pl.ds(r0, 125)], bounce)
        pltpu.sync_copy(bounce, out_hbm.at[c, pl.ds(r0, 125)])
        return carry

    lax.fori_loop(0, 5, oc, 0)


def _gcn_body(g_hbm, src3_hbm, dst3_hbm, out_hbm,
              src_idx, dst_idx, rows2, big_buf, acc_sh, gsems):
    c = lax.axis_index("c")
    s = lax.axis_index("s")
    chunk = c * NS + s
    pltpu.sync_copy(src3_hbm.at[chunk], src_idx)
    pltpu.sync_copy(dst3_hbm.at[chunk], dst_idx)
    _acc_init(acc_sh, big_buf, s)
    plsc.subcore_barrier()
    pltpu.async_copy(g_hbm.at[src_idx.at[0]], rows2.at[0], gsems.at[0])

    def blk(j, carry):
        b = lax.rem(j, 2)
        nb = 1 - b

        @pl.when(j < NBLK - 1)
        def _():
            pltpu.async_copy(g_hbm.at[src_idx.at[j + 1]], rows2.at[nb],
                             gsems.at[nb])

        pltpu.make_async_copy(g_hbm.at[src_idx.at[j]], rows2.at[b],
                              gsems.at[b]).wait()
        pltpu.sync_copy(rows2.at[b], acc_sh.at[dst_idx.at[j]], add=True)
        return carry

    lax.fori_loop(0, NBLK, blk, 0)
    plsc.subcore_barrier()
    _acc_out(acc_sh, big_buf, out_hbm, c, s)


def _sc_gcn(g, src3, dst3):
    return pl.kernel(
        _gcn_body,
        out_type=jax.ShapeDtypeStruct((NC, N, D), jnp.float32),
        mesh=_mesh,
        compiler_params=_sc_params,
        scratch_types=[
            pltpu.VMEM((NBLK, B), jnp.int32),
            pltpu.VMEM((NBLK, B), jnp.int32),
            pltpu.VMEM((2, B, D), jnp.float32),
            pltpu.VMEM((125, D), jnp.float32),
            pltpu.VMEM_SHARED((N, D), jnp.float32),
            pltpu.SemaphoreType.DMA((2,)),
        ],
    )(g, src3, dst3)


def _gat_body(hh_hbm, p_hbm, q_hbm, pmax_hbm, src3_hbm, dst3_hbm,
              acc_out, s_out,
              src_idx, dst_idx, rows2, w_buf, p_v, q_v, pm_v, buf1,
              big_buf, acc_sh, s_sh, gsems):
    c = lax.axis_index("c")
    s = lax.axis_index("s")
    chunk = c * NS + s
    pltpu.sync_copy(src3_hbm.at[chunk], src_idx)
    pltpu.sync_copy(dst3_hbm.at[chunk], dst_idx)
    pltpu.sync_copy(p_hbm, p_v)
    pltpu.sync_copy(q_hbm, q_v)
    pltpu.sync_copy(pmax_hbm, pm_v)
    _acc_init(acc_sh, big_buf, s)
    _zero_vec(buf1, 63)

    @pl.when(s < 10)
    def _():
        pltpu.sync_copy(buf1.at[pl.ds(0, 1000)],
                        s_sh.at[pl.ds(s * 1000, 1000)])

    plsc.subcore_barrier()
    pmv = pm_v[...]
    pltpu.async_copy(hh_hbm.at[src_idx.at[0]], rows2.at[0], gsems.at[0])

    def blk(j, carry):
        b = lax.rem(j, 2)
        nb = 1 - b

        @pl.when(j < NBLK - 1)
        def _():
            pltpu.async_copy(hh_hbm.at[src_idx.at[j + 1]], rows2.at[nb],
                             gsems.at[nb])

        for k in range(B // 16):
            si = src_idx[j, pl.ds(k * 16, 16)]
            di = dst_idx[j, pl.ds(k * 16, 16)]
            pv = plsc.load_gather(p_v, [si])
            qv = plsc.load_gather(q_v, [di])
            z = pv + qv
            e = jnp.maximum(z, 0.2 * z)
            zu = pmv + qv
            u = jnp.maximum(zu, 0.2 * zu)
            w_buf[pl.ds(k * 16, 16)] = jnp.exp(e - u)

        pltpu.make_async_copy(hh_hbm.at[src_idx.at[j]], rows2.at[b],
                              gsems.at[b]).wait()

        @plsc.parallel_loop(0, B, unroll=4)
        def _(i):
            bw = plsc.load_gather(w_buf, [jnp.full((16,), i, jnp.int32)])
            for t in range(D // 16):
                rows2[b, i, pl.ds(t * 16, 16)] = (
                    rows2[b, i, pl.ds(t * 16, 16)] * bw)

        pltpu.sync_copy(rows2.at[b], acc_sh.at[dst_idx.at[j]], add=True)
        pltpu.sync_copy(w_buf, s_sh.at[dst_idx.at[j]], add=True)
        return carry

    lax.fori_loop(0, NBLK, blk, 0)
    plsc.subcore_barrier()
    _acc_out(acc_sh, big_buf, acc_out, c, s)

    @pl.when(s < 10)
    def _():
        pltpu.sync_copy(s_sh.at[pl.ds(s * 1000, 1000)],
                        buf1.at[pl.ds(0, 1000)])
        pltpu.sync_copy(buf1.at[pl.ds(0, 1000)],
                        s_out.at[pl.ds(c * N + s * 1000, 1000)])


def _sc_gat(hh, p, q, pmax, src3, dst3):
    return pl.kernel(
        _gat_body,
        out_type=(jax.ShapeDtypeStruct((NC, N, D), jnp.float32),
                  jax.ShapeDtypeStruct((NC * N,), jnp.float32)),
        mesh=_mesh,
        compiler_params=_sc_params,
        scratch_types=[
            pltpu.VMEM((NBLK, B), jnp.int32),
            pltpu.VMEM((NBLK, B), jnp.int32),
            pltpu.VMEM((2, B, D), jnp.float32),
            pltpu.VMEM((B,), jnp.float32),
            pltpu.VMEM((N,), jnp.float32),
            pltpu.VMEM((N,), jnp.float32),
            pltpu.VMEM((16,), jnp.float32),
            pltpu.VMEM((1008,), jnp.float32),
            pltpu.VMEM((125, D), jnp.float32),
            pltpu.VMEM_SHARED((N, D), jnp.float32),
            pltpu.VMEM_SHARED((N,), jnp.float32),
            pltpu.SemaphoreType.DMA((2,)),
        ],
    )(hh, p, q, pmax, src3, dst3)


def _tc1_body(x_ref, w1_ref, da_ref, db_ref, g1_ref, dinv_ref):
    deg = da_ref[...] + db_ref[...] + 1.0
    dv = lax.rsqrt(deg)
    g1_ref[...] = jnp.dot(x_ref[...], w1_ref[...],
                          preferred_element_type=jnp.float32) * dv
    dinv_ref[...] = dv


def _tc1(x, W1, dA, dB):
    return pl.pallas_call(
        _tc1_body,
        grid=(NG,),
        in_specs=[
            pl.BlockSpec((R, DIN), lambda i: (i, 0)),
            pl.BlockSpec((DIN, D), lambda i: (0, 0)),
            pl.BlockSpec((R, 1), lambda i: (i, 0)),
            pl.BlockSpec((R, 1), lambda i: (i, 0)),
        ],
        out_specs=[
            pl.BlockSpec((R, D), lambda i: (i, 0)),
            pl.BlockSpec((R, 1), lambda i: (i, 0)),
        ],
        out_shape=[
            jax.ShapeDtypeStruct((N, D), jnp.float32),
            jax.ShapeDtypeStruct((N, 1), jnp.float32),
        ],
    )(x, W1, dA, dB)


def _tc2_body(accp_ref, g1_ref, dinv_ref, b1_ref, w2_ref, as_ref, ad_ref,
              hh_ref, p_ref, q_ref, pmax_ref):
    i = pl.program_id(0)
    acc = accp_ref[0] + accp_ref[1] + g1_ref[...]
    h1 = jnp.maximum(dinv_ref[...] * acc + b1_ref[...], 0.0)
    hh = jnp.dot(h1, w2_ref[...], preferred_element_type=jnp.float32)
    hh_ref[...] = hh
    p = jnp.dot(hh, as_ref[...], preferred_element_type=jnp.float32)
    q = jnp.dot(hh, ad_ref[...], preferred_element_type=jnp.float32)
    p_ref[...] = p
    q_ref[...] = q
    pb = jnp.max(p, axis=(0, 1), keepdims=True)

    @pl.when(i == 0)
    def _():
        pmax_ref[...] = pb

    @pl.when(i > 0)
    def _():
        pmax_ref[...] = jnp.maximum(pmax_ref[...], pb)


def _tc2(accP, g1, dinv, b1, W2, aS, aD):
    return pl.pallas_call(
        _tc2_body,
        grid=(NG,),
        in_specs=[
            pl.BlockSpec((NC, R, D), lambda i: (0, i, 0)),
            pl.BlockSpec((R, D), lambda i: (i, 0)),
            pl.BlockSpec((R, 1), lambda i: (i, 0)),
            pl.BlockSpec((1, D), lambda i: (0, 0)),
            pl.BlockSpec((D, D), lambda i: (0, 0)),
            pl.BlockSpec((D, 1), lambda i: (0, 0)),
            pl.BlockSpec((D, 1), lambda i: (0, 0)),
        ],
        out_specs=[
            pl.BlockSpec((R, D), lambda i: (i, 0)),
            pl.BlockSpec((R, 1), lambda i: (i, 0)),
            pl.BlockSpec((R, 1), lambda i: (i, 0)),
            pl.BlockSpec((1, 1), lambda i: (0, 0)),
        ],
        out_shape=[
            jax.ShapeDtypeStruct((N, D), jnp.float32),
            jax.ShapeDtypeStruct((N, 1), jnp.float32),
            jax.ShapeDtypeStruct((N, 1), jnp.float32),
            jax.ShapeDtypeStruct((1, 1), jnp.float32),
        ],
    )(accP, g1, dinv, b1, W2, aS, aD)


def _tc3_body(accp_ref, sp_ref, hh_ref, p_ref, q_ref, pmax_ref, dinv_ref,
              b2_ref, w3_ref, g3_ref):
    p = p_ref[...]
    q = q_ref[...]
    z = p + q
    e_self = jnp.maximum(z, 0.2 * z)
    zu = pmax_ref[0, 0] + q
    u = jnp.maximum(zu, 0.2 * zu)
    w_self = jnp.exp(e_self - u)
    den = sp_ref[0] + sp_ref[1] + w_self + 1e-16
    num = accp_ref[0] + accp_ref[1] + w_self * hh_ref[...]
    h2 = jnp.maximum(num / den + b2_ref[...], 0.0)
    g3_ref[...] = jnp.dot(h2, w3_ref[...],
                          preferred_element_type=jnp.float32) * dinv_ref[...]


def _tc3(accP, sP, hh, p, q, pmax, dinv, b2, W3):
    return pl.pallas_call(
        _tc3_body,
        grid=(NG,),
        in_specs=[
            pl.BlockSpec((NC, R, D), lambda i: (0, i, 0)),
            pl.BlockSpec((NC, R, 1), lambda i: (0, i, 0)),
            pl.BlockSpec((R, D), lambda i: (i, 0)),
            pl.BlockSpec((R, 1), lambda i: (i, 0)),
            pl.BlockSpec((R, 1), lambda i: (i, 0)),
            pl.BlockSpec((1, 1), lambda i: (0, 0)),
            pl.BlockSpec((R, 1), lambda i: (i, 0)),
            pl.BlockSpec((1, D), lambda i: (0, 0)),
            pl.BlockSpec((D, D), lambda i: (0, 0)),
        ],
        out_specs=[pl.BlockSpec((R, D), lambda i: (i, 0))],
        out_shape=[jax.ShapeDtypeStruct((N, D), jnp.float32)],
    )(accP, sP, hh, p, q, pmax, dinv, b2, W3)


def _tc4_body(accp_ref, g3_ref, dinv_ref, b3_ref, wo_ref, bo_ref,
              wb1_ref, bb1_ref, wb2_ref, bb2_ref,
              opt_ref, bt_ref, ge_ref):
    i = pl.program_id(0)
    acc = accp_ref[0] + accp_ref[1] + g3_ref[...]
    h3 = jnp.maximum(dinv_ref[...] * acc + b3_ref[...], 0.0)
    opt_ref[...] = jnp.dot(h3, wo_ref[...],
                           preferred_element_type=jnp.float32) + bo_ref[...]
    t = jnp.maximum(jnp.dot(h3, wb1_ref[...],
                            preferred_element_type=jnp.float32) + bb1_ref[...],
                    0.0)
    bt_ref[...] = jax.nn.sigmoid(
        jnp.dot(t, wb2_ref[...], preferred_element_type=jnp.float32)
        + bb2_ref[...])
    tot = jnp.sum(h3, axis=0, keepdims=True)

    @pl.when(i == 0)
    def _():
        ge_ref[...] = tot

    @pl.when(i > 0)
    def _():
        ge_ref[...] = ge_ref[...] + tot

    @pl.when(i == NG - 1)
    def _():
        ge_ref[...] = ge_ref[...] * (1.0 / N)


def _tc4(accP, g3, dinv, b3, Wo, bo, Wb1, bb1, Wb2, bb2):
    return pl.pallas_call(
        _tc4_body,
        grid=(NG,),
        in_specs=[
            pl.BlockSpec((NC, R, D), lambda i: (0, i, 0)),
            pl.BlockSpec((R, D), lambda i: (i, 0)),
            pl.BlockSpec((R, 1), lambda i: (i, 0)),
            pl.BlockSpec((1, D), lambda i: (0, 0)),
            pl.BlockSpec((D, NCLS), lambda i: (0, 0)),
            pl.BlockSpec((1, NCLS), lambda i: (0, 0)),
            pl.BlockSpec((D, 32), lambda i: (0, 0)),
            pl.BlockSpec((1, 32), lambda i: (0, 0)),
            pl.BlockSpec((32, 1), lambda i: (0, 0)),
            pl.BlockSpec((1, 1), lambda i: (0, 0)),
        ],
        out_specs=[
            pl.BlockSpec((R, NCLS), lambda i: (i, 0)),
            pl.BlockSpec((R, 1), lambda i: (i, 0)),
            pl.BlockSpec((1, D), lambda i: (0, 0)),
        ],
        out_shape=[
            jax.ShapeDtypeStruct((N, NCLS), jnp.float32),
            jax.ShapeDtypeStruct((N, 1), jnp.float32),
            jax.ShapeDtypeStruct((1, D), jnp.float32),
        ],
    )(accP, g3, dinv, b3, Wo, bo, Wb1, bb1, Wb2, bb2)


def kernel(x, edge_index, W1, b1, W2, a_src, a_dst, b2, W3, b3, Wo, bo,
           Wb1, bb1, Wb2, bb2):
    src3 = edge_index[0].reshape(NW, NBLK, B)
    dst3 = edge_index[1].reshape(NW, NBLK, B)

    degP = _sc_deg(dst3).reshape(NC, N)
    dA = degP[0].reshape(N, 1)
    dB = degP[1].reshape(N, 1)

    g1, dinv = _tc1(x, W1, dA, dB)
    acc1 = _sc_gcn(g1, src3, dst3)
    hh, p, q, pmax = _tc2(acc1, g1, dinv, b1.reshape(1, D), W2,
                          a_src.reshape(D, 1), a_dst.reshape(D, 1))

    pmax16 = jnp.broadcast_to(pmax.reshape(1), (16,))
    acc2, s2 = _sc_gat(hh, p.reshape(N), q.reshape(N), pmax16,
                       src3, dst3)
    g3 = _tc3(acc2, s2.reshape(NC, N, 1), hh, p, q, pmax, dinv,
              b2.reshape(1, D), W3)[0]

    acc3 = _sc_gcn(g3, src3, dst3)
    opt, bt, ge = _tc4(acc3, g3, dinv, b3.reshape(1, D), Wo,
                       bo.reshape(1, NCLS), Wb1, bb1.reshape(1, 32),
                       Wb2, bb2.reshape(1, 1))
    return opt, bt, ge.reshape(D)


# speedup vs baseline: 40.7409x; 1.0759x over previous
import functools

import jax
import jax.numpy as jnp
from jax import lax
from jax.experimental import pallas as pl
from jax.experimental.pallas import tpu as pltpu
from jax.experimental.pallas import tpu_sc as plsc

N = 10000
E = 320000
DIN = 128
D = 64
NCLS = 10

NC = 2
NS = 16
NW = NC * NS
EPT = E // NW
B = 80
NBLK = EPT // B

KB = 4

R = 1000
NG = N // R

_mesh = plsc.VectorSubcoreMesh(
    core_axis_name="c", subcore_axis_name="s", num_cores=NC, num_subcores=NS)
_sc_params = pltpu.CompilerParams(use_tc_tiling_on_sc=False,
                                  needs_layout_passes=False)


def _zero_vec(buf, nv):
    def zb(t, carry):
        buf[pl.ds(t * 16, 16)] = jnp.zeros((16,), jnp.float32)
        return carry
    lax.fori_loop(0, nv, zb, 0)


def _deg_body(dst3_hbm, out_hbm, dst_idx, ones_v, buf1, deg_sh, sem):
    c = lax.axis_index("c")
    s = lax.axis_index("s")
    chunk = c * NS + s
    pltpu.sync_copy(dst3_hbm.at[chunk], dst_idx)
    for k in range(B // 16):
        ones_v[pl.ds(k * 16, 16)] = jnp.ones((16,), jnp.float32)
    _zero_vec(buf1, 63)

    @pl.when(s < 10)
    def _():
        pltpu.sync_copy(buf1.at[pl.ds(0, 1000)],
                        deg_sh.at[pl.ds(s * 1000, 1000)])

    plsc.subcore_barrier()

    def blk(j, carry):
        pltpu.sync_copy(ones_v, deg_sh.at[dst_idx.at[j]], add=True)
        return carry

    lax.fori_loop(0, NBLK, blk, 0)
    plsc.subcore_barrier()

    @pl.when(s < 10)
    def _():
        pltpu.sync_copy(deg_sh.at[pl.ds(s * 1000, 1000)],
                        buf1.at[pl.ds(0, 1000)])
        pltpu.sync_copy(buf1.at[pl.ds(0, 1000)],
                        out_hbm.at[pl.ds(c * N + s * 1000, 1000)])


def _sc_deg(dst3):
    return pl.kernel(
        _deg_body,
        out_type=jax.ShapeDtypeStruct((NC * N,), jnp.float32),
        mesh=_mesh,
        compiler_params=_sc_params,
        scratch_types=[
            pltpu.VMEM((NBLK, B), jnp.int32),
            pltpu.VMEM((B,), jnp.float32),
            pltpu.VMEM((1008,), jnp.float32),
            pltpu.VMEM_SHARED((N,), jnp.float32),
            pltpu.SemaphoreType.DMA,
        ],
    )(dst3)


def _zero_rows(buf, nrows):
    def zb(r, carry):
        for t in range(D // 16):
            buf[r, pl.ds(t * 16, 16)] = jnp.zeros((16,), jnp.float32)
        return carry
    lax.fori_loop(0, nrows, zb, 0)


def _acc_init(acc_sh, bounce, s):
    _zero_rows(bounce, 125)

    def zc(ch, carry):
        pltpu.sync_copy(bounce, acc_sh.at[pl.ds(s * 625 + ch * 125, 125)])
        return carry

    lax.fori_loop(0, 5, zc, 0)


def _acc_out(acc_sh, bounce, out_hbm, c, s):
    def oc(ch, carry):
        r0 = s * 625 + ch * 125
        pltpu.sync_copy(acc_sh.at[pl.ds(r0, 125)], bounce)
        pltpu.sync_copy(bounce, out_hbm.at[c, pl.ds(r0, 125)])
        return carry

    lax.fori_loop(0, 5, oc, 0)


def _gcn_body(g_hbm, src3_hbm, dst3_hbm, out_hbm,
              src_idx, dst_idx, rowsK, big_buf, acc_sh, gsems, ssems):
    c = lax.axis_index("c")
    s = lax.axis_index("s")
    chunk = c * NS + s
    pltpu.sync_copy(src3_hbm.at[chunk], src_idx)
    pltpu.sync_copy(dst3_hbm.at[chunk], dst_idx)
    _acc_init(acc_sh, big_buf, s)
    plsc.subcore_barrier()
    pltpu.async_copy(g_hbm.at[src_idx.at[0]], rowsK.at[0], gsems.at[0])
    pltpu.async_copy(g_hbm.at[src_idx.at[1]], rowsK.at[1], gsems.at[1])

    def blk(j, carry):
        b = lax.rem(j, KB)
        pltpu.make_async_copy(g_hbm.at[src_idx.at[j]], rowsK.at[b],
                              gsems.at[b]).wait()
        pltpu.async_copy(rowsK.at[b], acc_sh.at[dst_idx.at[j]], ssems.at[b],
                         add=True)

        @pl.when(j + 2 < NBLK)
        def _():
            b2 = lax.rem(j + 2, KB)

            @pl.when(j >= 2)
            def _():
                pltpu.make_async_copy(rowsK.at[b2],
                                      acc_sh.at[dst_idx.at[j - 2]],
                                      ssems.at[b2]).wait()

            pltpu.async_copy(g_hbm.at[src_idx.at[j + 2]], rowsK.at[b2],
                             gsems.at[b2])

        return carry

    lax.fori_loop(0, NBLK, blk, 0)

    def drain(t, carry):
        jj = NBLK - KB + t
        bb = lax.rem(jj, KB)
        pltpu.make_async_copy(rowsK.at[bb], acc_sh.at[dst_idx.at[jj]],
                              ssems.at[bb]).wait()
        return carry

    lax.fori_loop(0, KB, drain, 0)
    plsc.subcore_barrier()
    _acc_out(acc_sh, big_buf, out_hbm, c, s)


def _sc_gcn(g, src3, dst3):
    return pl.kernel(
        _gcn_body,
        out_type=jax.ShapeDtypeStruct((NC, N, D), jnp.float32),
        mesh=_mesh,
        compiler_params=_sc_params,
        scratch_types=[
            pltpu.VMEM((NBLK, B), jnp.int32),
            pltpu.VMEM((NBLK, B), jnp.int32),
            pltpu.VMEM((KB, B, D), jnp.float32),
            pltpu.VMEM((125, D), jnp.float32),
            pltpu.VMEM_SHARED((N, D), jnp.float32),
            pltpu.SemaphoreType.DMA((KB,)),
            pltpu.SemaphoreType.DMA((KB,)),
        ],
    )(g, src3, dst3)


def _gat_body(hh_hbm, p_hbm, q_hbm, pmax_hbm, src3_hbm, dst3_hbm,
              acc_out, s_out,
              src_idx, dst_idx, rowsK, w_buf, p_v, q_v, pm_v, buf1,
              big_buf, acc_sh, s_sh, gsems, ssems):
    c = lax.axis_index("c")
    s = lax.axis_index("s")
    chunk = c * NS + s
    pltpu.sync_copy(src3_hbm.at[chunk], src_idx)
    pltpu.sync_copy(dst3_hbm.at[chunk], dst_idx)
    pltpu.sync_copy(p_hbm, p_v)
    pltpu.sync_copy(q_hbm, q_v)
    pltpu.sync_copy(pmax_hbm, pm_v)
    _acc_init(acc_sh, big_buf, s)
    _zero_vec(buf1, 63)

    @pl.when(s < 10)
    def _():
        pltpu.sync_copy(buf1.at[pl.ds(0, 1000)],
                        s_sh.at[pl.ds(s * 1000, 1000)])

    plsc.subcore_barrier()
    pmv = pm_v[...]
    pltpu.async_copy(hh_hbm.at[src_idx.at[0]], rowsK.at[0], gsems.at[0])
    pltpu.async_copy(hh_hbm.at[src_idx.at[1]], rowsK.at[1], gsems.at[1])

    def blk(j, carry):
        b = lax.rem(j, KB)
        for k in range(B // 16):
            si = src_idx[j, pl.ds(k * 16, 16)]
            di = dst_idx[j, pl.ds(k * 16, 16)]
            pv = plsc.load_gather(p_v, [si])
            qv = plsc.load_gather(q_v, [di])
            z = pv + qv
            e = jnp.maximum(z, 0.2 * z)
            zu = pmv + qv
            u = jnp.maximum(zu, 0.2 * zu)
            w_buf[pl.ds(k * 16, 16)] = jnp.exp(e - u)

        pltpu.make_async_copy(hh_hbm.at[src_idx.at[j]], rowsK.at[b],
                              gsems.at[b]).wait()

        @plsc.parallel_loop(0, B, unroll=4)
        def _(i):
            bw = plsc.load_gather(w_buf, [jnp.full((16,), i, jnp.int32)])
            for t in range(D // 16):
                rowsK[b, i, pl.ds(t * 16, 16)] = (
                    rowsK[b, i, pl.ds(t * 16, 16)] * bw)

        pltpu.async_copy(rowsK.at[b], acc_sh.at[dst_idx.at[j]], ssems.at[b],
                         add=True)
        pltpu.sync_copy(w_buf, s_sh.at[dst_idx.at[j]], add=True)

        @pl.when(j + 2 < NBLK)
        def _():
            b2 = lax.rem(j + 2, KB)

            @pl.when(j >= 2)
            def _():
                pltpu.make_async_copy(rowsK.at[b2],
                                      acc_sh.at[dst_idx.at[j - 2]],
                                      ssems.at[b2]).wait()

            pltpu.async_copy(hh_hbm.at[src_idx.at[j + 2]], rowsK.at[b2],
                             gsems.at[b2])

        return carry

    lax.fori_loop(0, NBLK, blk, 0)

    def drain(t, carry):
        jj = NBLK - KB + t
        bb = lax.rem(jj, KB)
        pltpu.make_async_copy(rowsK.at[bb], acc_sh.at[dst_idx.at[jj]],
                              ssems.at[bb]).wait()
        return carry

    lax.fori_loop(0, KB, drain, 0)
    plsc.subcore_barrier()
    _acc_out(acc_sh, big_buf, acc_out, c, s)

    @pl.when(s < 10)
    def _():
        pltpu.sync_copy(s_sh.at[pl.ds(s * 1000, 1000)],
                        buf1.at[pl.ds(0, 1000)])
        pltpu.sync_copy(buf1.at[pl.ds(0, 1000)],
                        s_out.at[pl.ds(c * N + s * 1000, 1000)])


def _sc_gat(hh, p, q, pmax, src3, dst3):
    return pl.kernel(
        _gat_body,
        out_type=(jax.ShapeDtypeStruct((NC, N, D), jnp.float32),
                  jax.ShapeDtypeStruct((NC * N,), jnp.float32)),
        mesh=_mesh,
        compiler_params=_sc_params,
        scratch_types=[
            pltpu.VMEM((NBLK, B), jnp.int32),
            pltpu.VMEM((NBLK, B), jnp.int32),
            pltpu.VMEM((KB, B, D), jnp.float32),
            pltpu.VMEM((B,), jnp.float32),
            pltpu.VMEM((N,), jnp.float32),
            pltpu.VMEM((N,), jnp.float32),
            pltpu.VMEM((16,), jnp.float32),
            pltpu.VMEM((1008,), jnp.float32),
            pltpu.VMEM((125, D), jnp.float32),
            pltpu.VMEM_SHARED((N, D), jnp.float32),
            pltpu.VMEM_SHARED((N,), jnp.float32),
            pltpu.SemaphoreType.DMA((KB,)),
            pltpu.SemaphoreType.DMA((KB,)),
        ],
    )(hh, p, q, pmax, src3, dst3)


def _tc1_body(x_ref, w1_ref, da_ref, db_ref, g1_ref, dinv_ref):
    deg = da_ref[...] + db_ref[...] + 1.0
    dv = lax.rsqrt(deg)
    g1_ref[...] = jnp.dot(x_ref[...], w1_ref[...],
                          preferred_element_type=jnp.float32) * dv
    dinv_ref[...] = dv


def _tc1(x, W1, dA, dB):
    return pl.pallas_call(
        _tc1_body,
        grid=(NG,),
        in_specs=[
            pl.BlockSpec((R, DIN), lambda i: (i, 0)),
            pl.BlockSpec((DIN, D), lambda i: (0, 0)),
            pl.BlockSpec((R, 1), lambda i: (i, 0)),
            pl.BlockSpec((R, 1), lambda i: (i, 0)),
        ],
        out_specs=[
            pl.BlockSpec((R, D), lambda i: (i, 0)),
            pl.BlockSpec((R, 1), lambda i: (i, 0)),
        ],
        out_shape=[
            jax.ShapeDtypeStruct((N, D), jnp.float32),
            jax.ShapeDtypeStruct((N, 1), jnp.float32),
        ],
    )(x, W1, dA, dB)


def _tc2_body(accp_ref, g1_ref, dinv_ref, b1_ref, w2_ref, as_ref, ad_ref,
              hh_ref, p_ref, q_ref, pmax_ref):
    i = pl.program_id(0)
    acc = accp_ref[0] + accp_ref[1] + g1_ref[...]
    h1 = jnp.maximum(dinv_ref[...] * acc + b1_ref[...], 0.0)
    hh = jnp.dot(h1, w2_ref[...], preferred_element_type=jnp.float32)
    hh_ref[...] = hh
    p = jnp.dot(hh, as_ref[...], preferred_element_type=jnp.float32)
    q = jnp.dot(hh, ad_ref[...], preferred_element_type=jnp.float32)
    p_ref[...] = p
    q_ref[...] = q
    pb = jnp.max(p, axis=(0, 1), keepdims=True)

    @pl.when(i == 0)
    def _():
        pmax_ref[...] = pb

    @pl.when(i > 0)
    def _():
        pmax_ref[...] = jnp.maximum(pmax_ref[...], pb)


def _tc2(accP, g1, dinv, b1, W2, aS, aD):
    return pl.pallas_call(
        _tc2_body,
        grid=(NG,),
        in_specs=[
            pl.BlockSpec((NC, R, D), lambda i: (0, i, 0)),
            pl.BlockSpec((R, D), lambda i: (i, 0)),
            pl.BlockSpec((R, 1), lambda i: (i, 0)),
            pl.BlockSpec((1, D), lambda i: (0, 0)),
            pl.BlockSpec((D, D), lambda i: (0, 0)),
            pl.BlockSpec((D, 1), lambda i: (0, 0)),
            pl.BlockSpec((D, 1), lambda i: (0, 0)),
        ],
        out_specs=[
            pl.BlockSpec((R, D), lambda i: (i, 0)),
            pl.BlockSpec((R, 1), lambda i: (i, 0)),
            pl.BlockSpec((R, 1), lambda i: (i, 0)),
            pl.BlockSpec((1, 1), lambda i: (0, 0)),
        ],
        out_shape=[
            jax.ShapeDtypeStruct((N, D), jnp.float32),
            jax.ShapeDtypeStruct((N, 1), jnp.float32),
            jax.ShapeDtypeStruct((N, 1), jnp.float32),
            jax.ShapeDtypeStruct((1, 1), jnp.float32),
        ],
    )(accP, g1, dinv, b1, W2, aS, aD)


def _tc3_body(accp_ref, sp_ref, hh_ref, p_ref, q_ref, pmax_ref, dinv_ref,
              b2_ref, w3_ref, g3_ref):
    p = p_ref[...]
    q = q_ref[...]
    z = p + q
    e_self = jnp.maximum(z, 0.2 * z)
    zu = pmax_ref[0, 0] + q
    u = jnp.maximum(zu, 0.2 * zu)
    w_self = jnp.exp(e_self - u)
    den = sp_ref[0] + sp_ref[1] + w_self + 1e-16
    num = accp_ref[0] + accp_ref[1] + w_self * hh_ref[...]
    h2 = jnp.maximum(num / den + b2_ref[...], 0.0)
    g3_ref[...] = jnp.dot(h2, w3_ref[...],
                          preferred_element_type=jnp.float32) * dinv_ref[...]


def _tc3(accP, sP, hh, p, q, pmax, dinv, b2, W3):
    return pl.pallas_call(
        _tc3_body,
        grid=(NG,),
        in_specs=[
            pl.BlockSpec((NC, R, D), lambda i: (0, i, 0)),
            pl.BlockSpec((NC, R, 1), lambda i: (0, i, 0)),
            pl.BlockSpec((R, D), lambda i: (i, 0)),
            pl.BlockSpec((R, 1), lambda i: (i, 0)),
            pl.BlockSpec((R, 1), lambda i: (i, 0)),
            pl.BlockSpec((1, 1), lambda i: (0, 0)),
            pl.BlockSpec((R, 1), lambda i: (i, 0)),
            pl.BlockSpec((1, D), lambda i: (0, 0)),
            pl.BlockSpec((D, D), lambda i: (0, 0)),
        ],
        out_specs=[pl.BlockSpec((R, D), lambda i: (i, 0))],
        out_shape=[jax.ShapeDtypeStruct((N, D), jnp.float32)],
    )(accP, sP, hh, p, q, pmax, dinv, b2, W3)


def _tc4_body(accp_ref, g3_ref, dinv_ref, b3_ref, wo_ref, bo_ref,
              wb1_ref, bb1_ref, wb2_ref, bb2_ref,
              opt_ref, bt_ref, ge_ref):
    i = pl.program_id(0)
    acc = accp_ref[0] + accp_ref[1] + g3_ref[...]
    h3 = jnp.maximum(dinv_ref[...] * acc + b3_ref[...], 0.0)
    opt_ref[...] = jnp.dot(h3, wo_ref[...],
                           preferred_element_type=jnp.float32) + bo_ref[...]
    t = jnp.maximum(jnp.dot(h3, wb1_ref[...],
                            preferred_element_type=jnp.float32) + bb1_ref[...],
                    0.0)
    bt_ref[...] = jax.nn.sigmoid(
        jnp.dot(t, wb2_ref[...], preferred_element_type=jnp.float32)
        + bb2_ref[...])
    tot = jnp.sum(h3, axis=0, keepdims=True)

    @pl.when(i == 0)
    def _():
        ge_ref[...] = tot

    @pl.when(i > 0)
    def _():
        ge_ref[...] = ge_ref[...] + tot

    @pl.when(i == NG - 1)
    def _():
        ge_ref[...] = ge_ref[...] * (1.0 / N)


def _tc4(accP, g3, dinv, b3, Wo, bo, Wb1, bb1, Wb2, bb2):
    return pl.pallas_call(
        _tc4_body,
        grid=(NG,),
        in_specs=[
            pl.BlockSpec((NC, R, D), lambda i: (0, i, 0)),
            pl.BlockSpec((R, D), lambda i: (i, 0)),
            pl.BlockSpec((R, 1), lambda i: (i, 0)),
            pl.BlockSpec((1, D), lambda i: (0, 0)),
            pl.BlockSpec((D, NCLS), lambda i: (0, 0)),
            pl.BlockSpec((1, NCLS), lambda i: (0, 0)),
            pl.BlockSpec((D, 32), lambda i: (0, 0)),
            pl.BlockSpec((1, 32), lambda i: (0, 0)),
            pl.BlockSpec((32, 1), lambda i: (0, 0)),
            pl.BlockSpec((1, 1), lambda i: (0, 0)),
        ],
        out_specs=[
            pl.BlockSpec((R, NCLS), lambda i: (i, 0)),
            pl.BlockSpec((R, 1), lambda i: (i, 0)),
            pl.BlockSpec((1, D), lambda i: (0, 0)),
        ],
        out_shape=[
            jax.ShapeDtypeStruct((N, NCLS), jnp.float32),
            jax.ShapeDtypeStruct((N, 1), jnp.float32),
            jax.ShapeDtypeStruct((1, D), jnp.float32),
        ],
    )(accP, g3, dinv, b3, Wo, bo, Wb1, bb1, Wb2, bb2)


def kernel(x, edge_index, W1, b1, W2, a_src, a_dst, b2, W3, b3, Wo, bo,
           Wb1, bb1, Wb2, bb2):
    src3 = edge_index[0].reshape(NW, NBLK, B)
    dst3 = edge_index[1].reshape(NW, NBLK, B)

    degP = _sc_deg(dst3).reshape(NC, N)
    dA = degP[0].reshape(N, 1)
    dB = degP[1].reshape(N, 1)

    g1, dinv = _tc1(x, W1, dA, dB)
    acc1 = _sc_gcn(g1, src3, dst3)
    hh, p, q, pmax = _tc2(acc1, g1, dinv, b1.reshape(1, D), W2,
                          a_src.reshape(D, 1), a_dst.reshape(D, 1))

    pmax16 = jnp.broadcast_to(pmax.reshape(1), (16,))
    acc2, s2 = _sc_gat(hh, p.reshape(N), q.reshape(N), pmax16,
                       src3, dst3)
    g3 = _tc3(acc2, s2.reshape(NC, N, 1), hh, p, q, pmax, dinv,
              b2.reshape(1, D), W3)[0]

    acc3 = _sc_gcn(g3, src3, dst3)
    opt, bt, ge = _tc4(acc3, g3, dinv, b3.reshape(1, D), Wo,
                       bo.reshape(1, NCLS), Wb1, bb1.reshape(1, 32),
                       Wb2, bb2.reshape(1, 1))
    return opt, bt, ge.reshape(D)
